# Initial kernel scaffold; baseline (speedup 1.0000x reference)
import jax
import jax.numpy as jnp
from jax.experimental import pallas as pl


def kernel(x, edge_index, W1l, W1r, b1, W2l, W2r, b2, W3l, W3r, b3):
    raise NotImplementedError("write your pallas kernel here")


# speedup vs baseline: 4.8365x; 4.8365x over previous
import functools

import jax
import jax.numpy as jnp
from jax import lax
from jax.experimental import pallas as pl
from jax.experimental.pallas import tpu as pltpu
from jax.experimental.pallas import tpu_sc as plsc

N_NODES = 10000
N_EDGES = 320000
NC = 2
NS = 16
EB = 80
EPT = N_EDGES // NS
NB = EPT // EB
NBT = 256
SLOTS = NBT * EB
SEG = 4000
NHALF = 5120
TRASH = 5120
ACCR = 5128
RPT = NHALF // NS
NPAD = 2 * NHALF


def _sc_mesh():
    return plsc.VectorSubcoreMesh(
        core_axis_name="c", subcore_axis_name="s", num_cores=NC, num_subcores=NS
    )


def _make_partition():
    out_type = [
        jax.ShapeDtypeStruct((NS, NBT, EB), jnp.int32),
        jax.ShapeDtypeStruct((NS, NBT, EB), jnp.int32),
        jax.ShapeDtypeStruct((NS * 16,), jnp.int32),
        jax.ShapeDtypeStruct((NS * NPAD,), jnp.float32),
    ]
    scratch = [
        pltpu.VMEM((SEG,), jnp.int32),
        pltpu.VMEM((SEG,), jnp.int32),
        pltpu.VMEM((NBT, EB), jnp.int32),
        pltpu.VMEM((NBT, EB), jnp.int32),
        pltpu.VMEM((NPAD,), jnp.float32),
        pltpu.VMEM((16,), jnp.int32),
    ]

    @functools.partial(pl.kernel, mesh=_sc_mesh(), out_type=out_type,
                       scratch_types=scratch,
                       compiler_params=pltpu.CompilerParams(
                           needs_layout_passes=False))
    def k(srcf, dstf, zsrc, tdst, zdeg,
          srcP, dstP, cnts, degP,
          segS, segD, bufS, bufD, deg, cntv):
        c = lax.axis_index("c")
        s = lax.axis_index("s")

        @pl.when(c == 0)
        def _():
            pltpu.sync_copy(zsrc, bufS)
            pltpu.sync_copy(tdst, bufD)

            def outer(ch, carry):
                base = pl.multiple_of(s * EPT + ch * SEG, 8)
                pltpu.sync_copy(srcf.at[pl.ds(base, SEG)], segS)
                pltpu.sync_copy(dstf.at[pl.ds(base, SEG)], segD)

                def inner(v, cr):
                    nA, nB = cr
                    d16 = segD[pl.ds(v * 16, 16)]
                    s16 = segS[pl.ds(v * 16, 16)]
                    mA = d16 < NHALF
                    mB = jnp.logical_not(mA)
                    iA = mA.astype(jnp.int32)
                    cA = plsc.cumsum(iA)
                    cB = plsc.cumsum(1 - iA)
                    pA = nA + cA - 1
                    pB = SLOTS - nB - cB
                    plsc.store_scatter(bufS, [pA // EB, pA % EB], s16, mask=mA)
                    plsc.store_scatter(bufD, [pA // EB, pA % EB], d16, mask=mA)
                    plsc.store_scatter(bufS, [pB // EB, pB % EB], s16, mask=mB)
                    plsc.store_scatter(bufD, [pB // EB, pB % EB],
                                       d16 - NHALF, mask=mB)
                    tA = jnp.sum(iA)
                    return nA + tA, nB + (16 - tA)

                return pl.loop(0, SEG // 16, init_carry=carry)(inner)

            nA, _ = pl.loop(0, EPT // SEG,
                            init_carry=(jnp.int32(0), jnp.int32(0)))(outer)

            pltpu.sync_copy(bufS, srcP.at[s])
            pltpu.sync_copy(bufD, dstP.at[s])
            cntv[...] = jnp.full((16,), nA, jnp.int32)
            pltpu.sync_copy(cntv, cnts.at[pl.ds(pl.multiple_of(s * 16, 8), 16)])

        @pl.when(c == 1)
        def _():
            pltpu.sync_copy(zdeg, deg)

            def outer(ch):
                base = pl.multiple_of(s * EPT + ch * SEG, 8)
                pltpu.sync_copy(dstf.at[pl.ds(base, SEG)], segD)

                def inner(v):
                    d16 = segD[pl.ds(v * 16, 16)]
                    cnt, mlast = plsc.scan_count(d16)
                    plsc.addupdate_scatter(deg, [d16],
                                           cnt.astype(jnp.float32), mask=mlast)

                pl.loop(0, SEG // 16)(inner)

            pl.loop(0, EPT // SEG)(outer)
            pltpu.sync_copy(
                deg, degP.at[pl.ds(pl.multiple_of(s * NPAD, 8), NPAD)])

    return k


def _zero_acc(acc, zrow, s):
    pltpu.sync_copy(zrow.at[pl.ds(0, RPT)], acc.at[pl.ds(s * RPT, RPT)])

    @pl.when(s == NS - 1)
    def _():
        pltpu.sync_copy(zrow.at[pl.ds(0, 8)], acc.at[pl.ds(NHALF, 8)])


def _chunk_ranges(cnts, cntv, s):
    pltpu.sync_copy(cnts.at[pl.ds(pl.multiple_of(s * 16, 8), 16)], cntv)
    nA = jnp.max(cntv[...])
    nbA = (nA + EB - 1) // EB
    jB0 = NBT - (EPT - nA + EB - 1) // EB
    return nbA, jB0


def _make_sc_agg_l1():
    out_type = [
        jax.ShapeDtypeStruct((NPAD, 128), jnp.float32),
        jax.ShapeDtypeStruct((NPAD, 128), jnp.float32),
    ]
    scratch = [
        pltpu.VMEM_SHARED((ACCR, 128), jnp.float32),
        pltpu.VMEM((NBT, EB), jnp.int32),
        pltpu.VMEM((NBT, EB), jnp.int32),
        pltpu.VMEM((EB, 128), jnp.float32),
        pltpu.VMEM((16,), jnp.int32),
        pltpu.SemaphoreType.DMA,
    ]

    @functools.partial(pl.kernel, mesh=_sc_mesh(), out_type=out_type,
                       scratch_types=scratch,
                       compiler_params=pltpu.CompilerParams(
                           needs_layout_passes=False))
    def k(h, srcP, dstP, cnts, zrow,
          p0, p1,
          acc, sidx, didx, rows, cntv, sem):
        c = lax.axis_index("c")
        s = lax.axis_index("s")

        pltpu.sync_copy(srcP.at[s], sidx)
        pltpu.sync_copy(dstP.at[s], didx)
        nbA, jB0 = _chunk_ranges(cnts, cntv, s)

        for lo, hi, row0 in ((0, nbA, 0), (jB0, NBT, NHALF)):
            mid = (lo + hi) // 2
            _zero_acc(acc, zrow, s)
            plsc.subcore_barrier()

            def edge_loop(jlo, jhi):
                @pl.loop(jlo, jhi)
                def _(j):
                    pltpu.async_copy(h.at[sidx.at[j]], rows, sem).wait()
                    pltpu.sync_copy(rows, acc.at[didx.at[j]], add=True)

            @pl.when(c == 0)
            def _():
                edge_loop(lo, mid)

            @pl.when(c == 1)
            def _():
                edge_loop(mid, hi)

            plsc.subcore_barrier()
            src_sl = pl.ds(s * RPT, RPT)
            dst_sl = pl.ds(row0 + s * RPT, RPT)

            @pl.when(c == 0)
            def _():
                pltpu.sync_copy(acc.at[src_sl], p0.at[dst_sl])

            @pl.when(c == 1)
            def _():
                pltpu.sync_copy(acc.at[src_sl], p1.at[dst_sl])

    return k


def _make_sc_agg_rest():
    out_type = [
        jax.ShapeDtypeStruct((NPAD, 128), jnp.float32),
        jax.ShapeDtypeStruct((NPAD, 128), jnp.float32),
    ]
    scratch = [
        pltpu.VMEM_SHARED((ACCR, 128), jnp.float32),
        pltpu.VMEM((NBT, EB), jnp.int32),
        pltpu.VMEM((NBT, EB), jnp.int32),
        pltpu.VMEM((EB, 128), jnp.float32),
        pltpu.VMEM((16,), jnp.int32),
        pltpu.SemaphoreType.DMA,
    ]

    @functools.partial(pl.kernel, mesh=_sc_mesh(), out_type=out_type,
                       scratch_types=scratch,
                       compiler_params=pltpu.CompilerParams(
                           needs_layout_passes=False))
    def k(h_lo, h_hi, srcP, dstP, cnts, zrow,
          o_lo, o_hi,
          acc, sidx, didx, rows, cntv, sem):
        c = lax.axis_index("c")
        s = lax.axis_index("s")

        pltpu.sync_copy(srcP.at[s], sidx)
        pltpu.sync_copy(dstP.at[s], didx)
        nbA, jB0 = _chunk_ranges(cnts, cntv, s)

        for lo, hi, row0 in ((0, nbA, 0), (jB0, NBT, NHALF)):
            _zero_acc(acc, zrow, s)
            plsc.subcore_barrier()

            def edge_loop(h_ref):
                @pl.loop(lo, hi)
                def _(j):
                    pltpu.async_copy(h_ref.at[sidx.at[j]], rows, sem).wait()
                    pltpu.sync_copy(rows, acc.at[didx.at[j]], add=True)

            @pl.when(c == 0)
            def _():
                edge_loop(h_lo)

            @pl.when(c == 1)
            def _():
                edge_loop(h_hi)

            plsc.subcore_barrier()
            src_sl = pl.ds(s * RPT, RPT)
            dst_sl = pl.ds(row0 + s * RPT, RPT)

            @pl.when(c == 0)
            def _():
                pltpu.sync_copy(acc.at[src_sl], o_lo.at[dst_sl])

            @pl.when(c == 1)
            def _():
                pltpu.sync_copy(acc.at[src_sl], o_hi.at[dst_sl])

    return k


def _tc_layer1(p0, p1, deg, x, Wl, Wr, b):
    n, din = x.shape
    dout = Wl.shape[1]
    R = 1000

    def body(p0_ref, p1_ref, d_ref, x_ref, wl_ref, wr_ref, b_ref, o_ref):
        cnt = jnp.sum(d_ref[...], axis=1)
        inv = 1.0 / jnp.maximum(cnt, 1.0)
        agg = (p0_ref[...] + p1_ref[...]) * inv[:, None]
        acc = jnp.dot(agg, wl_ref[...], preferred_element_type=jnp.float32)
        acc = acc + jnp.dot(x_ref[...], wr_ref[...],
                            preferred_element_type=jnp.float32)
        o_ref[...] = jnp.maximum(acc + b_ref[...], 0.0)

    return pl.pallas_call(
        body,
        grid=(n // R,),
        in_specs=[
            pl.BlockSpec((R, din), lambda i: (i, 0)),
            pl.BlockSpec((R, din), lambda i: (i, 0)),
            pl.BlockSpec((R, NS), lambda i: (i, 0)),
            pl.BlockSpec((R, din), lambda i: (i, 0)),
            pl.BlockSpec((din, dout), lambda i: (0, 0)),
            pl.BlockSpec((din, dout), lambda i: (0, 0)),
            pl.BlockSpec((1, dout), lambda i: (0, 0)),
        ],
        out_specs=pl.BlockSpec((R, dout), lambda i: (i, 0)),
        out_shape=jax.ShapeDtypeStruct((n, dout), jnp.float32),
    )(p0, p1, deg, x, Wl, Wr, b.reshape(1, dout))


def _tc_layer(a_lo, a_hi, deg, h, Wl, Wr, b, relu):
    n, din = h.shape
    dout = Wl.shape[1]
    R = 1000

    def body(al_ref, ah_ref, d_ref, h_ref, wl_ref, wr_ref, b_ref, o_ref):
        cnt = jnp.sum(d_ref[...], axis=1)
        inv = (1.0 / jnp.maximum(cnt, 1.0))[:, None]
        agg = jnp.concatenate([al_ref[...] * inv, ah_ref[...] * inv], axis=1)
        acc = jnp.dot(agg, wl_ref[...], preferred_element_type=jnp.float32)
        acc = acc + jnp.dot(h_ref[...], wr_ref[...],
                            preferred_element_type=jnp.float32)
        acc = acc + b_ref[...]
        if relu:
            acc = jnp.maximum(acc, 0.0)
        o_ref[...] = acc

    return pl.pallas_call(
        body,
        grid=(n // R,),
        in_specs=[
            pl.BlockSpec((R, 128), lambda i: (i, 0)),
            pl.BlockSpec((R, 128), lambda i: (i, 0)),
            pl.BlockSpec((R, NS), lambda i: (i, 0)),
            pl.BlockSpec((R, din), lambda i: (i, 0)),
            pl.BlockSpec((din, dout), lambda i: (0, 0)),
            pl.BlockSpec((din, dout), lambda i: (0, 0)),
            pl.BlockSpec((1, dout), lambda i: (0, 0)),
        ],
        out_specs=pl.BlockSpec((R, dout), lambda i: (i, 0)),
        out_shape=jax.ShapeDtypeStruct((n, dout), jnp.float32),
    )(a_lo, a_hi, deg, h, Wl, Wr, b.reshape(1, dout))


_partition = _make_partition()
_sc_l1 = _make_sc_agg_l1()
_sc_rest = _make_sc_agg_rest()


def kernel(x, edge_index, W1l, W1r, b1, W2l, W2r, b2, W3l, W3r, b3):
    srcf = edge_index[0].astype(jnp.int32)
    dstf = edge_index[1].astype(jnp.int32)

    zsrc = jnp.zeros((NBT, EB), jnp.int32)
    tdst = jnp.full((NBT, EB), TRASH, jnp.int32)
    zdeg = jnp.zeros((NPAD,), jnp.float32)
    zrow = jnp.zeros((RPT, 128), jnp.float32)

    srcP, dstP, cnts, degP = _partition(srcf, dstf, zsrc, tdst, zdeg)
    deg = degP.reshape(NS, NPAD)[:, :N_NODES].T

    p0, p1 = _sc_l1(x, srcP, dstP, cnts, zrow)
    h1 = _tc_layer1(p0[:N_NODES], p1[:N_NODES], deg, x, W1l, W1r, b1)

    a_lo, a_hi = _sc_rest(h1[:, :128], h1[:, 128:], srcP, dstP, cnts, zrow)
    h2 = _tc_layer(a_lo[:N_NODES], a_hi[:N_NODES], deg, h1,
                   W2l, W2r, b2, relu=True)

    a_lo, a_hi = _sc_rest(h2[:, :128], h2[:, 128:], srcP, dstP, cnts, zrow)
    out = _tc_layer(a_lo[:N_NODES], a_hi[:N_NODES], deg, h2,
                    W3l, W3r, b3, relu=False)
    return out


# speedup vs baseline: 7.1756x; 1.4836x over previous
import functools

import jax
import jax.numpy as jnp
from jax import lax
from jax.experimental import pallas as pl
from jax.experimental.pallas import tpu as pltpu
from jax.experimental.pallas import tpu_sc as plsc

N_NODES = 10000
N_EDGES = 320000
NC = 2
NS = 16
EB = 80
EPT = N_EDGES // NS
NB = EPT // EB
NBT = 256
SLOTS = NBT * EB
SEG = 4000
NHALF = 5120
TRASH = 5120
ACCR = 5128
RPT = NHALF // NS
NPAD = 2 * NHALF
RING = 4


def _sc_mesh():
    return plsc.VectorSubcoreMesh(
        core_axis_name="c", subcore_axis_name="s", num_cores=NC, num_subcores=NS
    )


def _make_partition():
    out_type = [
        jax.ShapeDtypeStruct((NS * SLOTS,), jnp.int32),
        jax.ShapeDtypeStruct((NS * SLOTS,), jnp.int32),
        jax.ShapeDtypeStruct((NS * 16,), jnp.int32),
        jax.ShapeDtypeStruct((NS * NPAD,), jnp.float32),
    ]
    scratch = [
        pltpu.VMEM((SEG,), jnp.int32),
        pltpu.VMEM((SEG,), jnp.int32),
        pltpu.VMEM((SLOTS,), jnp.int32),
        pltpu.VMEM((SLOTS,), jnp.int32),
        pltpu.VMEM((NPAD,), jnp.float32),
        pltpu.VMEM((16,), jnp.int32),
    ]

    @functools.partial(pl.kernel, mesh=_sc_mesh(), out_type=out_type,
                       scratch_types=scratch,
                       compiler_params=pltpu.CompilerParams(
                           needs_layout_passes=False))
    def k(srcf, dstf, zsrc, tdst, zdeg,
          srcP, dstP, cnts, degP,
          segS, segD, bufS, bufD, deg, cntv):
        c = lax.axis_index("c")
        s = lax.axis_index("s")

        @pl.when(c == 0)
        def _():
            pltpu.sync_copy(zsrc, bufS)
            pltpu.sync_copy(tdst, bufD)

            def outer(ch, carry):
                base = pl.multiple_of(s * EPT + ch * SEG, 8)
                pltpu.sync_copy(srcf.at[pl.ds(base, SEG)], segS)
                pltpu.sync_copy(dstf.at[pl.ds(base, SEG)], segD)

                def inner(v, cr):
                    nA, nB = cr
                    d16 = segD[pl.ds(v * 16, 16)]
                    s16 = segS[pl.ds(v * 16, 16)]
                    mA = d16 < NHALF
                    mB = jnp.logical_not(mA)
                    iA = mA.astype(jnp.int32)
                    cA = plsc.cumsum(iA)
                    cB = plsc.cumsum(1 - iA)
                    pA = nA + cA - 1
                    pB = SLOTS - nB - cB
                    plsc.store_scatter(bufS, [pA], s16, mask=mA)
                    plsc.store_scatter(bufD, [pA], d16, mask=mA)
                    plsc.store_scatter(bufS, [pB], s16, mask=mB)
                    plsc.store_scatter(bufD, [pB], d16 - NHALF, mask=mB)
                    tA = jnp.sum(iA)
                    return nA + tA, nB + (16 - tA)

                return pl.loop(0, SEG // 16, init_carry=carry)(inner)

            nA, _ = pl.loop(0, EPT // SEG,
                            init_carry=(jnp.int32(0), jnp.int32(0)))(outer)

            out_sl = pl.ds(pl.multiple_of(s * SLOTS, 8), SLOTS)
            pltpu.sync_copy(bufS, srcP.at[out_sl])
            pltpu.sync_copy(bufD, dstP.at[out_sl])
            cntv[...] = jnp.full((16,), nA, jnp.int32)
            pltpu.sync_copy(cntv, cnts.at[pl.ds(pl.multiple_of(s * 16, 8), 16)])

        @pl.when(c == 1)
        def _():
            pltpu.sync_copy(zdeg, deg)

            def outer(ch):
                base = pl.multiple_of(s * EPT + ch * SEG, 8)
                pltpu.sync_copy(dstf.at[pl.ds(base, SEG)], segD)

                def inner(v):
                    d16 = segD[pl.ds(v * 16, 16)]
                    cnt, mlast = plsc.scan_count(d16)
                    plsc.addupdate_scatter(deg, [d16],
                                           cnt.astype(jnp.float32), mask=mlast)

                pl.loop(0, SEG // 16)(inner)

            pl.loop(0, EPT // SEG)(outer)
            pltpu.sync_copy(
                deg, degP.at[pl.ds(pl.multiple_of(s * NPAD, 8), NPAD)])

    return k


def _zero_acc(acc, zrow, s):
    pltpu.sync_copy(zrow.at[pl.ds(0, RPT)], acc.at[pl.ds(s * RPT, RPT)])

    @pl.when(s == NS - 1)
    def _():
        pltpu.sync_copy(zrow.at[pl.ds(0, 8)], acc.at[pl.ds(NHALF, 8)])


def _chunk_ranges(cnts, cntv, s):
    pltpu.sync_copy(cnts.at[pl.ds(pl.multiple_of(s * 16, 8), 16)], cntv)
    nA = jnp.max(cntv[...])
    nbA = (nA + EB - 1) // EB
    jB0 = NBT - (EPT - nA + EB - 1) // EB
    return nbA, jB0


def _edge_loop(h_ref, srcP, dstP, acc, sidxr, didxr, rows,
               semI, semG, semS, base, lo, hi):
    ng = (hi - lo) // RING

    def group(g):
        j0 = lo + g * RING
        dI = []
        for b in range(RING):
            off = pl.multiple_of(base + (j0 + b) * EB, 8)
            dI.append((
                pltpu.async_copy(srcP.at[pl.ds(off, EB)], sidxr.at[b], semI),
                pltpu.async_copy(dstP.at[pl.ds(off, EB)], didxr.at[b], semI),
            ))
        dG = []
        for b in range(RING):
            dI[b][0].wait()
            dI[b][1].wait()
            dG.append(pltpu.async_copy(h_ref.at[sidxr.at[b]], rows.at[b],
                                       semG))
        dS = []
        for b in range(RING):
            dG[b].wait()
            dS.append(pltpu.async_copy(rows.at[b], acc.at[didxr.at[b]],
                                       semS, add=True))
        for d in dS:
            d.wait()

    pl.loop(0, ng)(group)

    tail = lo + ng * RING
    for t in range(RING - 1):
        j = tail + t

        @pl.when(j < hi)
        def _(j=j, t=t):
            off = pl.multiple_of(base + j * EB, 8)
            pltpu.sync_copy(srcP.at[pl.ds(off, EB)], sidxr.at[t])
            pltpu.sync_copy(dstP.at[pl.ds(off, EB)], didxr.at[t])
            pltpu.async_copy(h_ref.at[sidxr.at[t]], rows.at[t], semG).wait()
            pltpu.sync_copy(rows.at[t], acc.at[didxr.at[t]], add=True)


def _agg_scratch():
    return [
        pltpu.VMEM_SHARED((ACCR, 128), jnp.float32),
        pltpu.VMEM((RING, EB), jnp.int32),
        pltpu.VMEM((RING, EB), jnp.int32),
        pltpu.VMEM((RING, EB, 128), jnp.float32),
        pltpu.VMEM((16,), jnp.int32),
        pltpu.SemaphoreType.DMA,
        pltpu.SemaphoreType.DMA,
        pltpu.SemaphoreType.DMA,
    ]


def _make_sc_agg_l1():
    out_type = [
        jax.ShapeDtypeStruct((NPAD, 128), jnp.float32),
        jax.ShapeDtypeStruct((NPAD, 128), jnp.float32),
    ]

    @functools.partial(pl.kernel, mesh=_sc_mesh(), out_type=out_type,
                       scratch_types=_agg_scratch(),
                       compiler_params=pltpu.CompilerParams(
                           needs_layout_passes=False))
    def k(h, srcP, dstP, cnts, zrow,
          p0, p1,
          acc, sidxr, didxr, rows, cntv, semI, semG, semS):
        c = lax.axis_index("c")
        s = lax.axis_index("s")

        nbA, jB0 = _chunk_ranges(cnts, cntv, s)
        base = s * SLOTS

        for lo, hi, row0 in ((0, nbA, 0), (jB0, NBT, NHALF)):
            mid = (lo + hi) // 2
            _zero_acc(acc, zrow, s)
            plsc.subcore_barrier()

            def edge_loop(jlo, jhi):
                _edge_loop(h, srcP, dstP, acc, sidxr, didxr, rows,
                           semI, semG, semS, base, jlo, jhi)

            @pl.when(c == 0)
            def _():
                edge_loop(lo, mid)

            @pl.when(c == 1)
            def _():
                edge_loop(mid, hi)

            plsc.subcore_barrier()
            src_sl = pl.ds(s * RPT, RPT)
            dst_sl = pl.ds(row0 + s * RPT, RPT)

            @pl.when(c == 0)
            def _():
                pltpu.sync_copy(acc.at[src_sl], p0.at[dst_sl])

            @pl.when(c == 1)
            def _():
                pltpu.sync_copy(acc.at[src_sl], p1.at[dst_sl])

    return k


def _make_sc_agg_rest():
    out_type = [
        jax.ShapeDtypeStruct((NPAD, 128), jnp.float32),
        jax.ShapeDtypeStruct((NPAD, 128), jnp.float32),
    ]

    @functools.partial(pl.kernel, mesh=_sc_mesh(), out_type=out_type,
                       scratch_types=_agg_scratch(),
                       compiler_params=pltpu.CompilerParams(
                           needs_layout_passes=False))
    def k(h_lo, h_hi, srcP, dstP, cnts, zrow,
          o_lo, o_hi,
          acc, sidxr, didxr, rows, cntv, semI, semG, semS):
        c = lax.axis_index("c")
        s = lax.axis_index("s")

        nbA, jB0 = _chunk_ranges(cnts, cntv, s)
        base = s * SLOTS

        for lo, hi, row0 in ((0, nbA, 0), (jB0, NBT, NHALF)):
            _zero_acc(acc, zrow, s)
            plsc.subcore_barrier()

            def edge_loop(h_ref):
                _edge_loop(h_ref, srcP, dstP, acc, sidxr, didxr, rows,
                           semI, semG, semS, base, lo, hi)

            @pl.when(c == 0)
            def _():
                edge_loop(h_lo)

            @pl.when(c == 1)
            def _():
                edge_loop(h_hi)

            plsc.subcore_barrier()
            src_sl = pl.ds(s * RPT, RPT)
            dst_sl = pl.ds(row0 + s * RPT, RPT)

            @pl.when(c == 0)
            def _():
                pltpu.sync_copy(acc.at[src_sl], o_lo.at[dst_sl])

            @pl.when(c == 1)
            def _():
                pltpu.sync_copy(acc.at[src_sl], o_hi.at[dst_sl])

    return k


def _tc_layer1(p0, p1, deg, x, Wl, Wr, b):
    n, din = x.shape
    dout = Wl.shape[1]
    R = 1000

    def body(p0_ref, p1_ref, d_ref, x_ref, wl_ref, wr_ref, b_ref, o_ref):
        cnt = jnp.sum(d_ref[...], axis=1)
        inv = 1.0 / jnp.maximum(cnt, 1.0)
        agg = (p0_ref[...] + p1_ref[...]) * inv[:, None]
        acc = jnp.dot(agg, wl_ref[...], preferred_element_type=jnp.float32)
        acc = acc + jnp.dot(x_ref[...], wr_ref[...],
                            preferred_element_type=jnp.float32)
        o_ref[...] = jnp.maximum(acc + b_ref[...], 0.0)

    return pl.pallas_call(
        body,
        grid=(n // R,),
        in_specs=[
            pl.BlockSpec((R, din), lambda i: (i, 0)),
            pl.BlockSpec((R, din), lambda i: (i, 0)),
            pl.BlockSpec((R, NS), lambda i: (i, 0)),
            pl.BlockSpec((R, din), lambda i: (i, 0)),
            pl.BlockSpec((din, dout), lambda i: (0, 0)),
            pl.BlockSpec((din, dout), lambda i: (0, 0)),
            pl.BlockSpec((1, dout), lambda i: (0, 0)),
        ],
        out_specs=pl.BlockSpec((R, dout), lambda i: (i, 0)),
        out_shape=jax.ShapeDtypeStruct((n, dout), jnp.float32),
    )(p0, p1, deg, x, Wl, Wr, b.reshape(1, dout))


def _tc_layer(a_lo, a_hi, deg, h, Wl, Wr, b, relu):
    n, din = h.shape
    dout = Wl.shape[1]
    R = 1000

    def body(al_ref, ah_ref, d_ref, h_ref, wl_ref, wr_ref, b_ref, o_ref):
        cnt = jnp.sum(d_ref[...], axis=1)
        inv = (1.0 / jnp.maximum(cnt, 1.0))[:, None]
        agg = jnp.concatenate([al_ref[...] * inv, ah_ref[...] * inv], axis=1)
        acc = jnp.dot(agg, wl_ref[...], preferred_element_type=jnp.float32)
        acc = acc + jnp.dot(h_ref[...], wr_ref[...],
                            preferred_element_type=jnp.float32)
        acc = acc + b_ref[...]
        if relu:
            acc = jnp.maximum(acc, 0.0)
        o_ref[...] = acc

    return pl.pallas_call(
        body,
        grid=(n // R,),
        in_specs=[
            pl.BlockSpec((R, 128), lambda i: (i, 0)),
            pl.BlockSpec((R, 128), lambda i: (i, 0)),
            pl.BlockSpec((R, NS), lambda i: (i, 0)),
            pl.BlockSpec((R, din), lambda i: (i, 0)),
            pl.BlockSpec((din, dout), lambda i: (0, 0)),
            pl.BlockSpec((din, dout), lambda i: (0, 0)),
            pl.BlockSpec((1, dout), lambda i: (0, 0)),
        ],
        out_specs=pl.BlockSpec((R, dout), lambda i: (i, 0)),
        out_shape=jax.ShapeDtypeStruct((n, dout), jnp.float32),
    )(a_lo, a_hi, deg, h, Wl, Wr, b.reshape(1, dout))


_partition = _make_partition()
_sc_l1 = _make_sc_agg_l1()
_sc_rest = _make_sc_agg_rest()


def kernel(x, edge_index, W1l, W1r, b1, W2l, W2r, b2, W3l, W3r, b3):
    srcf = edge_index[0].astype(jnp.int32)
    dstf = edge_index[1].astype(jnp.int32)

    zsrc = jnp.zeros((SLOTS,), jnp.int32)
    tdst = jnp.full((SLOTS,), TRASH, jnp.int32)
    zdeg = jnp.zeros((NPAD,), jnp.float32)
    zrow = jnp.zeros((RPT, 128), jnp.float32)

    srcP, dstP, cnts, degP = _partition(srcf, dstf, zsrc, tdst, zdeg)
    deg = degP.reshape(NS, NPAD)[:, :N_NODES].T

    p0, p1 = _sc_l1(x, srcP, dstP, cnts, zrow)
    h1 = _tc_layer1(p0[:N_NODES], p1[:N_NODES], deg, x, W1l, W1r, b1)

    a_lo, a_hi = _sc_rest(h1[:, :128], h1[:, 128:], srcP, dstP, cnts, zrow)
    h2 = _tc_layer(a_lo[:N_NODES], a_hi[:N_NODES], deg, h1,
                   W2l, W2r, b2, relu=True)

    a_lo, a_hi = _sc_rest(h2[:, :128], h2[:, 128:], srcP, dstP, cnts, zrow)
    out = _tc_layer(a_lo[:N_NODES], a_hi[:N_NODES], deg, h2,
                    W3l, W3r, b3, relu=False)
    return out


# speedup vs baseline: 7.3726x; 1.0275x over previous
import functools

import jax
import jax.numpy as jnp
from jax import lax
from jax.experimental import pallas as pl
from jax.experimental.pallas import tpu as pltpu
from jax.experimental.pallas import tpu_sc as plsc

N_NODES = 10000
N_EDGES = 320000
NC = 2
NS = 16
EB = 80
EPT = N_EDGES // NS
NB = EPT // EB
NBT = 256
SLOTS = NBT * EB
SEG = 4000
NHALF = 5120
TRASH = 5120
ACCR = 5128
RPT = NHALF // NS
NPAD = 2 * NHALF
RING = 6


def _sc_mesh():
    return plsc.VectorSubcoreMesh(
        core_axis_name="c", subcore_axis_name="s", num_cores=NC, num_subcores=NS
    )


def _make_partition():
    out_type = [
        jax.ShapeDtypeStruct((NS * SLOTS,), jnp.int32),
        jax.ShapeDtypeStruct((NS * SLOTS,), jnp.int32),
        jax.ShapeDtypeStruct((NS * 16,), jnp.int32),
        jax.ShapeDtypeStruct((NS * NPAD,), jnp.float32),
    ]
    scratch = [
        pltpu.VMEM((SEG,), jnp.int32),
        pltpu.VMEM((SEG,), jnp.int32),
        pltpu.VMEM((SLOTS,), jnp.int32),
        pltpu.VMEM((SLOTS,), jnp.int32),
        pltpu.VMEM((NPAD,), jnp.float32),
        pltpu.VMEM((16,), jnp.int32),
    ]

    @functools.partial(pl.kernel, mesh=_sc_mesh(), out_type=out_type,
                       scratch_types=scratch,
                       compiler_params=pltpu.CompilerParams(
                           needs_layout_passes=False))
    def k(srcf, dstf, zsrc, tdst, zdeg,
          srcP, dstP, cnts, degP,
          segS, segD, bufS, bufD, deg, cntv):
        c = lax.axis_index("c")
        s = lax.axis_index("s")

        @pl.when(c == 0)
        def _():
            pltpu.sync_copy(zsrc, bufS)
            pltpu.sync_copy(tdst, bufD)

            def outer(ch, carry):
                base = pl.multiple_of(s * EPT + ch * SEG, 8)
                pltpu.sync_copy(srcf.at[pl.ds(base, SEG)], segS)
                pltpu.sync_copy(dstf.at[pl.ds(base, SEG)], segD)

                def inner(v, cr):
                    nA, nB = cr
                    d16 = segD[pl.ds(v * 16, 16)]
                    s16 = segS[pl.ds(v * 16, 16)]
                    mA = d16 < NHALF
                    mB = jnp.logical_not(mA)
                    iA = mA.astype(jnp.int32)
                    cA = plsc.cumsum(iA)
                    cB = plsc.cumsum(1 - iA)
                    pA = nA + cA - 1
                    pB = SLOTS - nB - cB
                    plsc.store_scatter(bufS, [pA], s16, mask=mA)
                    plsc.store_scatter(bufD, [pA], d16, mask=mA)
                    plsc.store_scatter(bufS, [pB], s16, mask=mB)
                    plsc.store_scatter(bufD, [pB], d16 - NHALF, mask=mB)
                    tA = jnp.sum(iA)
                    return nA + tA, nB + (16 - tA)

                return pl.loop(0, SEG // 16, init_carry=carry)(inner)

            nA, _ = pl.loop(0, EPT // SEG,
                            init_carry=(jnp.int32(0), jnp.int32(0)))(outer)

            out_sl = pl.ds(pl.multiple_of(s * SLOTS, 8), SLOTS)
            pltpu.sync_copy(bufS, srcP.at[out_sl])
            pltpu.sync_copy(bufD, dstP.at[out_sl])
            cntv[...] = jnp.full((16,), nA, jnp.int32)
            pltpu.sync_copy(cntv, cnts.at[pl.ds(pl.multiple_of(s * 16, 8), 16)])

        @pl.when(c == 1)
        def _():
            pltpu.sync_copy(zdeg, deg)

            def outer(ch):
                base = pl.multiple_of(s * EPT + ch * SEG, 8)
                pltpu.sync_copy(dstf.at[pl.ds(base, SEG)], segD)

                def inner(v):
                    d16 = segD[pl.ds(v * 16, 16)]
                    cnt, mlast = plsc.scan_count(d16)
                    plsc.addupdate_scatter(deg, [d16],
                                           cnt.astype(jnp.float32), mask=mlast)

                pl.loop(0, SEG // 16)(inner)

            pl.loop(0, EPT // SEG)(outer)
            pltpu.sync_copy(
                deg, degP.at[pl.ds(pl.multiple_of(s * NPAD, 8), NPAD)])

    return k


def _zero_acc(acc, zrow, s):
    pltpu.sync_copy(zrow.at[pl.ds(0, RPT)], acc.at[pl.ds(s * RPT, RPT)])

    @pl.when(s == NS - 1)
    def _():
        pltpu.sync_copy(zrow.at[pl.ds(0, 8)], acc.at[pl.ds(NHALF, 8)])


def _chunk_ranges(cnts, cntv, s):
    pltpu.sync_copy(cnts.at[pl.ds(pl.multiple_of(s * 16, 8), 16)], cntv)
    nA = jnp.max(cntv[...])
    nbA = (nA + EB - 1) // EB
    jB0 = NBT - (EPT - nA + EB - 1) // EB
    return nbA, jB0


def _edge_loop(h_ref, srcP, dstP, acc, sidxr, didxr, rows,
               semI, semG, semS, base, lo, hi):
    ng = (hi - lo) // RING

    def group(g):
        j0 = lo + g * RING
        dI = []
        for b in range(RING):
            off = pl.multiple_of(base + (j0 + b) * EB, 8)
            dI.append((
                pltpu.async_copy(srcP.at[pl.ds(off, EB)], sidxr.at[b], semI),
                pltpu.async_copy(dstP.at[pl.ds(off, EB)], didxr.at[b], semI),
            ))
        dG = []
        for b in range(RING):
            dI[b][0].wait()
            dI[b][1].wait()
            dG.append(pltpu.async_copy(h_ref.at[sidxr.at[b]], rows.at[b],
                                       semG))
        dS = []
        for b in range(RING):
            dG[b].wait()
            dS.append(pltpu.async_copy(rows.at[b], acc.at[didxr.at[b]],
                                       semS, add=True))
        for d in dS:
            d.wait()

    pl.loop(0, ng)(group)

    tail = lo + ng * RING
    for t in range(RING - 1):
        j = tail + t

        @pl.when(j < hi)
        def _(j=j, t=t):
            off = pl.multiple_of(base + j * EB, 8)
            pltpu.sync_copy(srcP.at[pl.ds(off, EB)], sidxr.at[t])
            pltpu.sync_copy(dstP.at[pl.ds(off, EB)], didxr.at[t])
            pltpu.async_copy(h_ref.at[sidxr.at[t]], rows.at[t], semG).wait()
            pltpu.sync_copy(rows.at[t], acc.at[didxr.at[t]], add=True)


def _agg_scratch():
    return [
        pltpu.VMEM_SHARED((ACCR, 128), jnp.float32),
        pltpu.VMEM((RING, EB), jnp.int32),
        pltpu.VMEM((RING, EB), jnp.int32),
        pltpu.VMEM((RING, EB, 128), jnp.float32),
        pltpu.VMEM((16,), jnp.int32),
        pltpu.SemaphoreType.DMA,
        pltpu.SemaphoreType.DMA,
        pltpu.SemaphoreType.DMA,
    ]


def _make_sc_agg_l1():
    out_type = [
        jax.ShapeDtypeStruct((NPAD, 128), jnp.float32),
        jax.ShapeDtypeStruct((NPAD, 128), jnp.float32),
    ]

    @functools.partial(pl.kernel, mesh=_sc_mesh(), out_type=out_type,
                       scratch_types=_agg_scratch(),
                       compiler_params=pltpu.CompilerParams(
                           needs_layout_passes=False))
    def k(h, srcP, dstP, cnts, zrow,
          p0, p1,
          acc, sidxr, didxr, rows, cntv, semI, semG, semS):
        c = lax.axis_index("c")
        s = lax.axis_index("s")

        nbA, jB0 = _chunk_ranges(cnts, cntv, s)
        base = s * SLOTS

        for lo, hi, row0 in ((0, nbA, 0), (jB0, NBT, NHALF)):
            mid = (lo + hi) // 2
            _zero_acc(acc, zrow, s)
            plsc.subcore_barrier()

            def edge_loop(jlo, jhi):
                _edge_loop(h, srcP, dstP, acc, sidxr, didxr, rows,
                           semI, semG, semS, base, jlo, jhi)

            @pl.when(c == 0)
            def _():
                edge_loop(lo, mid)

            @pl.when(c == 1)
            def _():
                edge_loop(mid, hi)

            plsc.subcore_barrier()
            src_sl = pl.ds(s * RPT, RPT)
            dst_sl = pl.ds(row0 + s * RPT, RPT)

            @pl.when(c == 0)
            def _():
                pltpu.sync_copy(acc.at[src_sl], p0.at[dst_sl])

            @pl.when(c == 1)
            def _():
                pltpu.sync_copy(acc.at[src_sl], p1.at[dst_sl])

    return k


def _make_sc_agg_rest():
    out_type = [
        jax.ShapeDtypeStruct((NPAD, 128), jnp.float32),
        jax.ShapeDtypeStruct((NPAD, 128), jnp.float32),
    ]

    @functools.partial(pl.kernel, mesh=_sc_mesh(), out_type=out_type,
                       scratch_types=_agg_scratch(),
                       compiler_params=pltpu.CompilerParams(
                           needs_layout_passes=False))
    def k(h_lo, h_hi, srcP, dstP, cnts, zrow,
          o_lo, o_hi,
          acc, sidxr, didxr, rows, cntv, semI, semG, semS):
        c = lax.axis_index("c")
        s = lax.axis_index("s")

        nbA, jB0 = _chunk_ranges(cnts, cntv, s)
        base = s * SLOTS

        for lo, hi, row0 in ((0, nbA, 0), (jB0, NBT, NHALF)):
            _zero_acc(acc, zrow, s)
            plsc.subcore_barrier()

            def edge_loop(h_ref):
                _edge_loop(h_ref, srcP, dstP, acc, sidxr, didxr, rows,
                           semI, semG, semS, base, lo, hi)

            @pl.when(c == 0)
            def _():
                edge_loop(h_lo)

            @pl.when(c == 1)
            def _():
                edge_loop(h_hi)

            plsc.subcore_barrier()
            src_sl = pl.ds(s * RPT, RPT)
            dst_sl = pl.ds(row0 + s * RPT, RPT)

            @pl.when(c == 0)
            def _():
                pltpu.sync_copy(acc.at[src_sl], o_lo.at[dst_sl])

            @pl.when(c == 1)
            def _():
                pltpu.sync_copy(acc.at[src_sl], o_hi.at[dst_sl])

    return k


def _tc_layer1(p0, p1, deg, x, Wl, Wr, b):
    n, din = x.shape
    dout = Wl.shape[1]
    R = 1000

    def body(p0_ref, p1_ref, d_ref, x_ref, wl_ref, wr_ref, b_ref, o_ref):
        cnt = jnp.sum(d_ref[...], axis=1)
        inv = 1.0 / jnp.maximum(cnt, 1.0)
        agg = (p0_ref[...] + p1_ref[...]) * inv[:, None]
        acc = jnp.dot(agg, wl_ref[...], preferred_element_type=jnp.float32)
        acc = acc + jnp.dot(x_ref[...], wr_ref[...],
                            preferred_element_type=jnp.float32)
        o_ref[...] = jnp.maximum(acc + b_ref[...], 0.0)

    return pl.pallas_call(
        body,
        grid=(n // R,),
        in_specs=[
            pl.BlockSpec((R, din), lambda i: (i, 0)),
            pl.BlockSpec((R, din), lambda i: (i, 0)),
            pl.BlockSpec((R, NS), lambda i: (i, 0)),
            pl.BlockSpec((R, din), lambda i: (i, 0)),
            pl.BlockSpec((din, dout), lambda i: (0, 0)),
            pl.BlockSpec((din, dout), lambda i: (0, 0)),
            pl.BlockSpec((1, dout), lambda i: (0, 0)),
        ],
        out_specs=pl.BlockSpec((R, dout), lambda i: (i, 0)),
        out_shape=jax.ShapeDtypeStruct((n, dout), jnp.float32),
    )(p0, p1, deg, x, Wl, Wr, b.reshape(1, dout))


def _tc_layer(a_lo, a_hi, deg, h, Wl, Wr, b, relu):
    n, din = h.shape
    dout = Wl.shape[1]
    R = 1000

    def body(al_ref, ah_ref, d_ref, h_ref, wl_ref, wr_ref, b_ref, o_ref):
        cnt = jnp.sum(d_ref[...], axis=1)
        inv = (1.0 / jnp.maximum(cnt, 1.0))[:, None]
        agg = jnp.concatenate([al_ref[...] * inv, ah_ref[...] * inv], axis=1)
        acc = jnp.dot(agg, wl_ref[...], preferred_element_type=jnp.float32)
        acc = acc + jnp.dot(h_ref[...], wr_ref[...],
                            preferred_element_type=jnp.float32)
        acc = acc + b_ref[...]
        if relu:
            acc = jnp.maximum(acc, 0.0)
        o_ref[...] = acc

    return pl.pallas_call(
        body,
        grid=(n // R,),
        in_specs=[
            pl.BlockSpec((R, 128), lambda i: (i, 0)),
            pl.BlockSpec((R, 128), lambda i: (i, 0)),
            pl.BlockSpec((R, NS), lambda i: (i, 0)),
            pl.BlockSpec((R, din), lambda i: (i, 0)),
            pl.BlockSpec((din, dout), lambda i: (0, 0)),
            pl.BlockSpec((din, dout), lambda i: (0, 0)),
            pl.BlockSpec((1, dout), lambda i: (0, 0)),
        ],
        out_specs=pl.BlockSpec((R, dout), lambda i: (i, 0)),
        out_shape=jax.ShapeDtypeStruct((n, dout), jnp.float32),
    )(a_lo, a_hi, deg, h, Wl, Wr, b.reshape(1, dout))


_partition = _make_partition()
_sc_l1 = _make_sc_agg_l1()
_sc_rest = _make_sc_agg_rest()


def kernel(x, edge_index, W1l, W1r, b1, W2l, W2r, b2, W3l, W3r, b3):
    srcf = edge_index[0].astype(jnp.int32)
    dstf = edge_index[1].astype(jnp.int32)

    zsrc = jnp.zeros((SLOTS,), jnp.int32)
    tdst = jnp.full((SLOTS,), TRASH, jnp.int32)
    zdeg = jnp.zeros((NPAD,), jnp.float32)
    zrow = jnp.zeros((RPT, 128), jnp.float32)

    srcP, dstP, cnts, degP = _partition(srcf, dstf, zsrc, tdst, zdeg)
    deg = degP.reshape(NS, NPAD)[:, :N_NODES].T

    p0, p1 = _sc_l1(x, srcP, dstP, cnts, zrow)
    h1 = _tc_layer1(p0[:N_NODES], p1[:N_NODES], deg, x, W1l, W1r, b1)

    a_lo, a_hi = _sc_rest(h1[:, :128], h1[:, 128:], srcP, dstP, cnts, zrow)
    h2 = _tc_layer(a_lo[:N_NODES], a_hi[:N_NODES], deg, h1,
                   W2l, W2r, b2, relu=True)

    a_lo, a_hi = _sc_rest(h2[:, :128], h2[:, 128:], srcP, dstP, cnts, zrow)
    out = _tc_layer(a_lo[:N_NODES], a_hi[:N_NODES], deg, h2,
                    W3l, W3r, b3, relu=False)
    return out
